# baseline (speedup 1.0000x reference)
import functools

import jax
import jax.numpy as jnp
from jax import lax
from jax.experimental import pallas as pl
from jax.experimental.pallas import tpu as pltpu
from jax.experimental.pallas import tpu_sc as plsc

_NC = 2
_NS = 16
_NW = _NC * _NS


@functools.lru_cache(maxsize=None)
def _make_sc_gather(V, D, B):
    assert B % (8 * _NW) == 0 and D % 16 == 0
    b_per_w = B // _NW
    mesh = plsc.VectorSubcoreMesh(core_axis_name="c", subcore_axis_name="s")

    @functools.partial(
        pl.kernel,
        mesh=mesh,
        out_type=jax.ShapeDtypeStruct((B, D), jnp.float32),
        scratch_types=[
            pltpu.VMEM((b_per_w,), jnp.int32),
            pltpu.VMEM((b_per_w, D), jnp.float32),
            pltpu.SemaphoreType.DMA,
        ],
        compiler_params=pltpu.CompilerParams(use_tc_tiling_on_sc=False),
    )
    def gather_kernel(table_hbm, idx_hbm, out_hbm, idx_v, rows_v, sem):
        wid = lax.axis_index("s") * _NC + lax.axis_index("c")
        base = wid * b_per_w
        pltpu.sync_copy(idx_hbm.at[pl.ds(base, b_per_w)], idx_v)
        pltpu.async_copy(table_hbm.at[idx_v], rows_v, sem).wait()
        pltpu.sync_copy(rows_v, out_hbm.at[pl.ds(base, b_per_w)])

    return gather_kernel


def _lstm_body(x_ref, wih_ref, whh_ref, b_ref, wfc_ref, bfc_ref, out_ref,
               h_scr, c_scr):
    T = x_ref.shape[0]
    H = whh_ref.shape[0]
    h_scr[...] = jnp.zeros_like(h_scr)
    c_scr[...] = jnp.zeros_like(c_scr)

    def step(t, carry):
        xt = x_ref[t]
        gates = (
            jnp.dot(xt, wih_ref[...], preferred_element_type=jnp.float32)
            + jnp.dot(h_scr[...], whh_ref[...],
                      preferred_element_type=jnp.float32)
            + b_ref[...]
        )
        i = jax.nn.sigmoid(gates[:, :H])
        f = jax.nn.sigmoid(gates[:, H:2 * H])
        g = jnp.tanh(gates[:, 2 * H:3 * H])
        o = jax.nn.sigmoid(gates[:, 3 * H:])
        c = f * c_scr[...] + i * g
        c_scr[...] = c
        h_scr[...] = o * jnp.tanh(c)
        return carry

    lax.fori_loop(0, T, step, 0)
    out_ref[...] = (
        jnp.dot(h_scr[...], wfc_ref[...], preferred_element_type=jnp.float32)
        + bfc_ref[...]
    )


def kernel(text, emb, W_ih, W_hh, b_ih, b_hh, W_fc, b_fc):
    T, B = text.shape
    V, E = emb.shape
    H = W_hh.shape[1]
    NC = W_fc.shape[0]

    idx = text.reshape(T * B)
    x_flat = _make_sc_gather(V, E, T * B)(emb, idx)
    x = x_flat.reshape(T, B, E)

    wih_t = W_ih.T
    whh_t = W_hh.T
    bias = (b_ih + b_hh).reshape(1, 4 * H)
    NCP = 128
    wfc_t = jnp.zeros((H, NCP), jnp.float32).at[:, :NC].set(W_fc.T)
    bfc = jnp.zeros((1, NCP), jnp.float32).at[:, :NC].set(b_fc)

    out = pl.pallas_call(
        _lstm_body,
        out_shape=jax.ShapeDtypeStruct((B, NCP), jnp.float32),
        scratch_shapes=[
            pltpu.VMEM((B, H), jnp.float32),
            pltpu.VMEM((B, H), jnp.float32),
        ],
    )(x, wih_t, whh_t, bias, wfc_t, bfc)
    return out[:, :NC]
